# baseline (speedup 1.0000x reference)
import jax
import jax.numpy as jnp
from jax.experimental import pallas as pl
from jax.experimental.pallas import tpu as pltpu

B, S = 512, 512
V, E, H = 50000, 128, 128
NB = 2
BB = B // NB
T = 32
NC = S // T


def _gru_body(wd_ref, idxt_ref, emb_ref, wcat_ref, bcat_ref, wo_ref,
              out_ref, esc_a, esc_b, wd_a, wd_b, sems):
    iota8 = jax.lax.broadcasted_iota(jnp.int32, (8, E), 0)
    m0 = (iota8 & 1) == 0
    m1 = (iota8 & 2) == 0
    m2 = (iota8 & 4) == 0

    def start_dma(c, wd_sm, sem):
        c0 = pl.multiple_of(c * T, 8)
        pltpu.make_async_copy(wd_ref.at[pl.ds(c0, T), :], wd_sm, sem).start()

    def wait_dma(wd_sm, sem):
        pltpu.make_async_copy(wd_sm, wd_sm, sem).wait()

    def gather_t(c, t, wd_sm, esc_w):
        idxt = idxt_ref[0, c * T + t]

        def vreg8(g8):
            idxv = jnp.broadcast_to(idxt[:, g8:g8 + 1], (8, E))
            p = []
            for k in range(4):
                w = wd_sm[t, g8 * 4 + k]
                for q13 in (w & 0x1FFF, w >> 13):
                    rows8 = emb_ref[pl.ds(pl.multiple_of(q13 * 8, 8), 8), :]
                    p.append(jnp.take_along_axis(rows8, idxv, axis=0))
            q = [jnp.where(m0, p[2 * k], p[2 * k + 1]) for k in range(4)]
            r0 = jnp.where(m1, q[0], q[1])
            r1 = jnp.where(m1, q[2], q[3])
            return jnp.where(m2, r0, r1)
        for g in range(BB // 16):
            acc = jnp.concatenate([vreg8(2 * g), vreg8(2 * g + 1)], axis=0)
            hi = acc.astype(jnp.bfloat16)
            lo = (acc - hi.astype(jnp.float32)).astype(jnp.bfloat16)
            row0 = pl.multiple_of(t * BB + g * 16, 16)
            esc_w[pl.ds(row0, 16), 0:E] = hi
            esc_w[pl.ds(row0, 16), E:2 * E] = hi
            esc_w[pl.ds(row0, 16), 2 * E:3 * E] = lo

    def run_chunk(c, h, esc_r, esc_w, wd_n, sem_n, wd_f, sem_f):
        wait_dma(wd_n, sem_n)
        cg = jnp.minimum(c + 1, NC - 1)

        @pl.when(c < NC - 1)
        def _():
            start_dma(jnp.minimum(c + 2, NC - 1), wd_f, sem_f)

        def step_one(t, h):
            gather_t(cg, t, wd_n, esc_w)
            row0 = pl.multiple_of(t * BB, 8)
            ecat = esc_r[pl.ds(row0, BB), :]
            h_hi = h.astype(jnp.bfloat16)
            h_lo = (h - h_hi.astype(jnp.float32)).astype(jnp.bfloat16)
            lhs = jnp.concatenate([ecat, h_hi, h_hi, h_lo], axis=1)
            g = jnp.dot(lhs, wcat_ref[:], preferred_element_type=jnp.float32)
            rz = jax.nn.sigmoid(g[:, :2 * H] + bcat_ref[:, :2 * H])
            r = rz[:, :H]
            z = rz[:, H:]
            xn = g[:, 2 * H:3 * H] + bcat_ref[:, 2 * H:3 * H]
            hn = g[:, 3 * H:] + bcat_ref[:, 3 * H:]
            n = jnp.tanh(xn + r * hn)
            return (1.0 - z) * n + z * h

        def step4(i, h):
            for u in range(4):
                h = step_one(4 * i + u, h)
            return h
        return jax.lax.fori_loop(0, T // 4, step4, h)

    start_dma(0, wd_a, sems.at[0])
    wait_dma(wd_a, sems.at[0])
    jax.lax.fori_loop(0, T, lambda t, u: (gather_t(0, t, wd_a, esc_a), 0)[1], 0)
    start_dma(1, wd_b, sems.at[1])

    def chunk2(m, h):
        c0 = 2 * m
        h = run_chunk(c0, h, esc_a, esc_b, wd_b, sems.at[1], wd_a, sems.at[0])
        h = run_chunk(c0 + 1, h, esc_b, esc_a, wd_a, sems.at[0],
                      wd_b, sems.at[1])
        return h

    h0 = jnp.zeros((BB, H), jnp.float32)
    hT = jax.lax.fori_loop(0, NC // 2, chunk2, h0)

    hT_hi = hT.astype(jnp.bfloat16)
    hT_lo = (hT - hT_hi.astype(jnp.float32)).astype(jnp.bfloat16)
    hT_cat = jnp.concatenate([hT_hi, hT_hi, hT_lo], axis=1)
    out_ref[:] = jnp.dot(hT_cat, wo_ref[:], preferred_element_type=jnp.float32)


def _hilo_rows(w):
    hi = w.astype(jnp.bfloat16)
    lo = (w - hi.astype(jnp.float32)).astype(jnp.bfloat16)
    return jnp.concatenate([hi, lo, hi], axis=0)


def kernel(x, emb, w_ih, w_hh, b_ih, b_hh, w_out, b_out):
    xT = x.T.astype(jnp.int32)
    xq = xT >> 3
    wd = xq[:, 0::2] | (xq[:, 1::2] << 13)
    am = (xT & 7).T.reshape(NB, BB // 8, 8, S)
    idxt = jnp.zeros((NB, S, 8, 128), jnp.int32)
    idxt = idxt.at[:, :, :, :BB // 8].set(am.transpose(0, 3, 2, 1))

    wi3 = _hilo_rows(w_ih.T.astype(jnp.float32))
    wh3 = _hilo_rows(w_hh.T.astype(jnp.float32))
    wcat = jnp.zeros((3 * E + 3 * H, 4 * H), jnp.bfloat16)
    wcat = wcat.at[:3 * E, :2 * H].set(wi3[:, :2 * H])
    wcat = wcat.at[:3 * E, 2 * H:3 * H].set(wi3[:, 2 * H:])
    wcat = wcat.at[3 * E:, :2 * H].set(wh3[:, :2 * H])
    wcat = wcat.at[3 * E:, 3 * H:].set(wh3[:, 2 * H:])
    bcat = jnp.concatenate([(b_ih + b_hh)[:2 * H], b_ih[2 * H:],
                            b_hh[2 * H:]]).reshape(1, 4 * H).astype(jnp.float32)
    wo = _hilo_rows(jnp.zeros((H, 128), jnp.float32).at[:, :2].set(w_out.T))

    out = pl.pallas_call(
        _gru_body,
        grid=(NB,),
        in_specs=[
            pl.BlockSpec((S, BB // 2), lambda i: (0, i)),
            pl.BlockSpec((1, S, 8, 128), lambda i: (i, 0, 0, 0)),
            pl.BlockSpec((V, E), lambda i: (0, 0)),
            pl.BlockSpec((3 * E + 3 * H, 4 * H), lambda i: (0, 0)),
            pl.BlockSpec((1, 4 * H), lambda i: (0, 0)),
            pl.BlockSpec((3 * H, 128), lambda i: (0, 0)),
        ],
        out_specs=pl.BlockSpec((BB, 128), lambda i: (i, 0)),
        out_shape=jax.ShapeDtypeStruct((B, 128), jnp.float32),
        scratch_shapes=[
            pltpu.VMEM((T * BB, 3 * E), jnp.bfloat16),
            pltpu.VMEM((T * BB, 3 * E), jnp.bfloat16),
            pltpu.SMEM((T, BB // 2), jnp.int32),
            pltpu.SMEM((T, BB // 2), jnp.int32),
            pltpu.SemaphoreType.DMA((2,)),
        ],
        compiler_params=pltpu.CompilerParams(
            dimension_semantics=(pltpu.PARALLEL,),
            vmem_limit_bytes=56 * 1024 * 1024,
        ),
    )(wd, idxt, emb, wcat, bcat, wo)
    return out[:, :2] + b_out


# speedup vs baseline: 1.0178x; 1.0178x over previous
import jax
import jax.numpy as jnp
from jax.experimental import pallas as pl
from jax.experimental.pallas import tpu as pltpu

B, S = 512, 512
V, E, H = 50000, 128, 128
NB = 2
BB = B // NB
T = 16
NC = S // T


def _gru_body(wd_ref, idxt_ref, emb_ref, wcat_ref, bcat_ref, wo_ref,
              out_ref, esc_a, esc_b, wd_a, wd_b, sems):
    iota8 = jax.lax.broadcasted_iota(jnp.int32, (8, E), 0)
    m0 = (iota8 & 1) == 0
    m1 = (iota8 & 2) == 0
    m2 = (iota8 & 4) == 0

    def start_dma(c, wd_sm, sem):
        c0 = pl.multiple_of(c * T, 8)
        pltpu.make_async_copy(wd_ref.at[pl.ds(c0, T), :], wd_sm, sem).start()

    def wait_dma(wd_sm, sem):
        pltpu.make_async_copy(wd_sm, wd_sm, sem).wait()

    def gather_t(c, t, wd_sm, esc_w):
        idxt = idxt_ref[0, c * T + t]

        def vreg8(g8):
            idxv = jnp.broadcast_to(idxt[:, g8:g8 + 1], (8, E))
            p = []
            for k in range(4):
                w = wd_sm[t, g8 * 4 + k]
                for q13 in (w & 0x1FFF, w >> 13):
                    rows8 = emb_ref[pl.ds(pl.multiple_of(q13 * 8, 8), 8), :]
                    p.append(jnp.take_along_axis(rows8, idxv, axis=0))
            q = [jnp.where(m0, p[2 * k], p[2 * k + 1]) for k in range(4)]
            r0 = jnp.where(m1, q[0], q[1])
            r1 = jnp.where(m1, q[2], q[3])
            return jnp.where(m2, r0, r1)
        for g in range(BB // 16):
            acc = jnp.concatenate([vreg8(2 * g), vreg8(2 * g + 1)], axis=0)
            hi = acc.astype(jnp.bfloat16)
            lo = (acc - hi.astype(jnp.float32)).astype(jnp.bfloat16)
            row0 = pl.multiple_of(t * BB + g * 16, 16)
            esc_w[pl.ds(row0, 16), 0:E] = hi
            esc_w[pl.ds(row0, 16), E:2 * E] = hi
            esc_w[pl.ds(row0, 16), 2 * E:3 * E] = lo

    def run_chunk(c, h, esc_r, esc_w, wd_n, sem_n, wd_f, sem_f):
        wait_dma(wd_n, sem_n)
        cg = jnp.minimum(c + 1, NC - 1)

        @pl.when(c < NC - 1)
        def _():
            start_dma(jnp.minimum(c + 2, NC - 1), wd_f, sem_f)

        def step_one(t, h):
            gather_t(cg, t, wd_n, esc_w)
            row0 = pl.multiple_of(t * BB, 8)
            ecat = esc_r[pl.ds(row0, BB), :]
            h_hi = h.astype(jnp.bfloat16)
            h_lo = (h - h_hi.astype(jnp.float32)).astype(jnp.bfloat16)
            lhs = jnp.concatenate([ecat, h_hi, h_hi, h_lo], axis=1)
            g = jnp.dot(lhs, wcat_ref[:], preferred_element_type=jnp.float32)
            rz = jax.nn.sigmoid(g[:, :2 * H] + bcat_ref[:, :2 * H])
            r = rz[:, :H]
            z = rz[:, H:]
            xn = g[:, 2 * H:3 * H] + bcat_ref[:, 2 * H:3 * H]
            hn = g[:, 3 * H:] + bcat_ref[:, 3 * H:]
            n = jnp.tanh(xn + r * hn)
            return (1.0 - z) * n + z * h

        def step4(i, h):
            for u in range(4):
                h = step_one(4 * i + u, h)
            return h
        return jax.lax.fori_loop(0, T // 4, step4, h)

    start_dma(0, wd_a, sems.at[0])
    wait_dma(wd_a, sems.at[0])
    jax.lax.fori_loop(0, T, lambda t, u: (gather_t(0, t, wd_a, esc_a), 0)[1], 0)
    start_dma(1, wd_b, sems.at[1])

    def chunk2(m, h):
        c0 = 2 * m
        h = run_chunk(c0, h, esc_a, esc_b, wd_b, sems.at[1], wd_a, sems.at[0])
        h = run_chunk(c0 + 1, h, esc_b, esc_a, wd_a, sems.at[0],
                      wd_b, sems.at[1])
        return h

    h0 = jnp.zeros((BB, H), jnp.float32)
    hT = jax.lax.fori_loop(0, NC // 2, chunk2, h0)

    hT_hi = hT.astype(jnp.bfloat16)
    hT_lo = (hT - hT_hi.astype(jnp.float32)).astype(jnp.bfloat16)
    hT_cat = jnp.concatenate([hT_hi, hT_hi, hT_lo], axis=1)
    out_ref[:] = jnp.dot(hT_cat, wo_ref[:], preferred_element_type=jnp.float32)


def _hilo_rows(w):
    hi = w.astype(jnp.bfloat16)
    lo = (w - hi.astype(jnp.float32)).astype(jnp.bfloat16)
    return jnp.concatenate([hi, lo, hi], axis=0)


def kernel(x, emb, w_ih, w_hh, b_ih, b_hh, w_out, b_out):
    xT = x.T.astype(jnp.int32)
    xq = xT >> 3
    wd = xq[:, 0::2] | (xq[:, 1::2] << 13)
    am = (xT & 7).T.reshape(NB, BB // 8, 8, S)
    idxt = jnp.zeros((NB, S, 8, 128), jnp.int32)
    idxt = idxt.at[:, :, :, :BB // 8].set(am.transpose(0, 3, 2, 1))

    wi3 = _hilo_rows(w_ih.T.astype(jnp.float32))
    wh3 = _hilo_rows(w_hh.T.astype(jnp.float32))
    wcat = jnp.zeros((3 * E + 3 * H, 4 * H), jnp.bfloat16)
    wcat = wcat.at[:3 * E, :2 * H].set(wi3[:, :2 * H])
    wcat = wcat.at[:3 * E, 2 * H:3 * H].set(wi3[:, 2 * H:])
    wcat = wcat.at[3 * E:, :2 * H].set(wh3[:, :2 * H])
    wcat = wcat.at[3 * E:, 3 * H:].set(wh3[:, 2 * H:])
    bcat = jnp.concatenate([(b_ih + b_hh)[:2 * H], b_ih[2 * H:],
                            b_hh[2 * H:]]).reshape(1, 4 * H).astype(jnp.float32)
    wo = _hilo_rows(jnp.zeros((H, 128), jnp.float32).at[:, :2].set(w_out.T))

    out = pl.pallas_call(
        _gru_body,
        grid=(NB,),
        in_specs=[
            pl.BlockSpec((S, BB // 2), lambda i: (0, i)),
            pl.BlockSpec((1, S, 8, 128), lambda i: (i, 0, 0, 0)),
            pl.BlockSpec((V, E), lambda i: (0, 0)),
            pl.BlockSpec((3 * E + 3 * H, 4 * H), lambda i: (0, 0)),
            pl.BlockSpec((1, 4 * H), lambda i: (0, 0)),
            pl.BlockSpec((3 * H, 128), lambda i: (0, 0)),
        ],
        out_specs=pl.BlockSpec((BB, 128), lambda i: (i, 0)),
        out_shape=jax.ShapeDtypeStruct((B, 128), jnp.float32),
        scratch_shapes=[
            pltpu.VMEM((T * BB, 3 * E), jnp.bfloat16),
            pltpu.VMEM((T * BB, 3 * E), jnp.bfloat16),
            pltpu.SMEM((T, BB // 2), jnp.int32),
            pltpu.SMEM((T, BB // 2), jnp.int32),
            pltpu.SemaphoreType.DMA((2,)),
        ],
        compiler_params=pltpu.CompilerParams(
            dimension_semantics=(pltpu.PARALLEL,),
            vmem_limit_bytes=56 * 1024 * 1024,
        ),
    )(wd, idxt, emb, wcat, bcat, wo)
    return out[:, :2] + b_out
